# baseline (speedup 1.0000x reference)
import jax
import jax.numpy as jnp
from jax import lax
from jax.experimental import pallas as pl
from jax.experimental.pallas import tpu as pltpu
from jax.experimental.pallas import tpu_sc as plsc

_MAX_REL = 128
_EMB = 64
_LEN = 2048
_TAB = 2 * _MAX_REL + 1
_EXT_PAD = 2 * _LEN + 8
_NC = 2
_NS = 16
_ROWS_PER_WORKER = _LEN // (_NC * _NS)


def _build_frev_kernel(w_ref, frev_ref):
    top = _LEN - _MAX_REL - 1
    frev_ref[0:top, :] = jnp.broadcast_to(
        w_ref[_TAB - 1:_TAB, :], (top, _EMB))
    frev_ref[top + _TAB:_EXT_PAD, :] = jnp.broadcast_to(
        w_ref[0:1, :], (_EXT_PAD - top - _TAB, _EMB))
    for j in range(_TAB):
        frev_ref[top + j:top + j + 1, :] = w_ref[_TAB - 1 - j:_TAB - j, :]


def _sc_stream_body(frev_hbm, out_hbm, frev_sh, sem):
    c = lax.axis_index("c")
    s = lax.axis_index("s")

    @pl.when(s == 0)
    def _stage():
        pltpu.sync_copy(frev_hbm, frev_sh)

    plsc.subcore_barrier()

    wid = s * _NC + c
    base_row = wid * _ROWS_PER_WORKER
    descs = []
    for t in range(_ROWS_PER_WORKER):
        row = base_row + t
        descs.append(pltpu.async_copy(
            frev_sh.at[pl.ds(_LEN - 1 - row, _LEN), :],
            out_hbm.at[row],
            sem,
        ))
    for d in descs:
        d.wait()


@jax.jit
def _run(W):
    frev = pl.pallas_call(
        _build_frev_kernel,
        in_specs=[pl.BlockSpec((_TAB, _EMB), lambda: (0, 0))],
        out_specs=pl.BlockSpec((_EXT_PAD, _EMB), lambda: (0, 0)),
        out_shape=jax.ShapeDtypeStruct((_EXT_PAD, _EMB), jnp.float32),
    )(W)

    sc_call = pl.kernel(
        _sc_stream_body,
        out_type=jax.ShapeDtypeStruct((_LEN, _LEN, _EMB), jnp.float32),
        mesh=plsc.VectorSubcoreMesh(
            core_axis_name="c", subcore_axis_name="s"),
        scratch_types=[
            pltpu.MemorySpace.VMEM_SHARED((_EXT_PAD, _EMB), jnp.float32),
            pltpu.SemaphoreType.DMA,
        ],
        compiler_params=pltpu.CompilerParams(use_tc_tiling_on_sc=True),
    )
    return sc_call(frev)


def kernel(W, length):
    return _run(W)


# speedup vs baseline: 1.2166x; 1.2166x over previous
import jax
import jax.numpy as jnp
from jax.experimental import pallas as pl
from jax.experimental.pallas import tpu as pltpu

_MAX_REL = 128
_EMB = 64
_LEN = 2048
_TAB = 2 * _MAX_REL + 1
_EXT_PAD = 2 * _LEN
_ROWS_PER_STEP = 32
_STEPS = _LEN // _ROWS_PER_STEP


def _rpe_kernel(w_ref, out_ref, frev_ref, sems):
    k = pl.program_id(0)

    @pl.when(k == 0)
    def _build():
        top = _LEN - _MAX_REL - 1
        frev_ref[0:top, :] = jnp.broadcast_to(
            w_ref[_TAB - 1:_TAB, :], (top, _EMB))
        frev_ref[top + _TAB:_EXT_PAD, :] = jnp.broadcast_to(
            w_ref[0:1, :], (_EXT_PAD - top - _TAB, _EMB))
        for j in range(_TAB):
            frev_ref[top + j:top + j + 1, :] = w_ref[_TAB - 1 - j:_TAB - j, :]

    def copy_for(row, bank, r):
        return pltpu.make_async_copy(
            frev_ref.at[pl.ds(_LEN - 1 - row, _LEN), :],
            out_ref.at[row],
            sems.at[bank, r],
        )

    bank = jax.lax.rem(k, 2)
    for r in range(_ROWS_PER_STEP):
        copy_for(k * _ROWS_PER_STEP + r, bank, r).start()

    @pl.when(k > 0)
    def _wait_prev():
        for r in range(_ROWS_PER_STEP):
            copy_for((k - 1) * _ROWS_PER_STEP + r, 1 - bank, r).wait()

    @pl.when(k == _STEPS - 1)
    def _wait_last():
        for r in range(_ROWS_PER_STEP):
            copy_for(k * _ROWS_PER_STEP + r, bank, r).wait()


@jax.jit
def _run(W):
    return pl.pallas_call(
        _rpe_kernel,
        grid=(_STEPS,),
        in_specs=[pl.BlockSpec((_TAB, _EMB), lambda i: (0, 0))],
        out_specs=pl.BlockSpec(memory_space=pl.ANY),
        out_shape=jax.ShapeDtypeStruct((_LEN, _LEN, _EMB), jnp.float32),
        scratch_shapes=[
            pltpu.VMEM((_EXT_PAD, _EMB), jnp.float32),
            pltpu.SemaphoreType.DMA((2, _ROWS_PER_STEP)),
        ],
    )(W)


def kernel(W, length):
    return _run(W)


# speedup vs baseline: 1.2503x; 1.0277x over previous
import jax
import jax.numpy as jnp
from jax.experimental import pallas as pl
from jax.experimental.pallas import tpu as pltpu

_MAX_REL = 128
_EMB = 64
_LEN = 2048
_TAB = 2 * _MAX_REL + 1
_ROWS_PER_STEP = 16
_STEPS = _LEN // _ROWS_PER_STEP
_W2 = 2 * _EMB


def _build_ab(w_ref, a_ref, b_ref):
    w256 = w_ref[_TAB - 1:_TAB, :]
    w0 = w_ref[0:1, :]
    cc256 = jnp.concatenate([w256, w256], axis=1)
    cc0 = jnp.concatenate([w0, w0], axis=1)
    a_ref[0:960, :] = jnp.broadcast_to(cc256, (960, _W2))
    b_ref[0:959, :] = jnp.broadcast_to(cc256, (959, _W2))
    a_ref[1088:_LEN, :] = jnp.broadcast_to(cc0, (960, _W2))
    b_ref[1087:_LEN, :] = jnp.broadcast_to(cc0, (961, _W2))
    for p in range(960, 1088):
        i1 = 256 - (2 * p - 1919)
        a_ref[p:p + 1, :] = jnp.concatenate(
            [w_ref[i1:i1 + 1, :], w_ref[i1 - 1:i1, :]], axis=1)
    for p in range(959, 1087):
        i1 = 256 - (2 * p + 1 - 1919)
        b_ref[p:p + 1, :] = jnp.concatenate(
            [w_ref[i1:i1 + 1, :], w_ref[i1 - 1:i1, :]], axis=1)


def _rpe_kernel(w_ref, out_ref, a_ref, b_ref, sems):
    k = pl.program_id(0)

    @pl.when(k == 0)
    def _build():
        _build_ab(w_ref, a_ref, b_ref)

    def copy_for(row, parity, bank, r):
        src = a_ref if parity else b_ref
        q = (_LEN - 1 - row) // 2
        return pltpu.make_async_copy(
            src.at[pl.ds(q, _LEN // 2), :],
            out_ref.at[row],
            sems.at[bank, r],
        )

    bank = jax.lax.rem(k, 2)
    for r in range(_ROWS_PER_STEP):
        copy_for(k * _ROWS_PER_STEP + r, r % 2, bank, r).start()

    @pl.when(k > 0)
    def _wait_prev():
        for r in range(_ROWS_PER_STEP):
            copy_for((k - 1) * _ROWS_PER_STEP + r, r % 2, 1 - bank, r).wait()

    @pl.when(k == _STEPS - 1)
    def _wait_last():
        for r in range(_ROWS_PER_STEP):
            copy_for(k * _ROWS_PER_STEP + r, r % 2, bank, r).wait()


@jax.jit
def _run(W):
    out = pl.pallas_call(
        _rpe_kernel,
        grid=(_STEPS,),
        in_specs=[pl.BlockSpec((_TAB, _EMB), lambda i: (0, 0))],
        out_specs=pl.BlockSpec(memory_space=pl.ANY),
        out_shape=jax.ShapeDtypeStruct((_LEN, _LEN // 2, _W2), jnp.float32),
        scratch_shapes=[
            pltpu.VMEM((_LEN, _W2), jnp.float32),
            pltpu.VMEM((_LEN, _W2), jnp.float32),
            pltpu.SemaphoreType.DMA((2, _ROWS_PER_STEP)),
        ],
    )(W)
    return out.reshape(_LEN, _LEN, _EMB)


def kernel(W, length):
    return _run(W)
